# Initial kernel scaffold; baseline (speedup 1.0000x reference)
import jax
import jax.numpy as jnp
from jax import lax
from jax.experimental import pallas as pl
from jax.experimental.pallas import tpu as pltpu
from jax.experimental.pallas import tpu_sc as plsc

N_NODES = 10000
N_PAD = 10240
B = 8
B_HALF = 4
E = 160000
TILES = 16
E_PER_TILE = E // TILES
VECS = E_PER_TILE // 16
ROWS = B_HALF * N_PAD // 16
ROWS_PER_B = N_PAD // 16
N_PER_TILE = N_PAD // TILES
FROWS = N_PER_TILE // 16


def _gnn_body(x_hbm, src_hbm, dst_hbm, w_hbm, bias_hbm, rowidx_hbm, zeros_hbm,
              out_hbm,
              x_v, acc_v, src_v, dst_v, w_v, rowidx_v, fbuf_v, obuf_v, bias_v,
              shared_acc):
    c = lax.axis_index("c")
    s = lax.axis_index("s")

    pltpu.sync_copy(zeros_hbm, acc_v)
    pltpu.sync_copy(x_hbm.at[pl.ds(c * B_HALF, B_HALF), :], x_v)
    e0 = s * E_PER_TILE
    pltpu.sync_copy(src_hbm.at[pl.ds(e0, E_PER_TILE)], src_v)
    pltpu.sync_copy(dst_hbm.at[pl.ds(e0, E_PER_TILE)], dst_v)
    pltpu.sync_copy(w_hbm.at[pl.ds(e0, E_PER_TILE)], w_v)
    pltpu.sync_copy(rowidx_hbm, rowidx_v)

    def step(i, carry):
        sl = pl.ds(i * 16, 16)
        sv = src_v[sl]
        dv = dst_v[sl]
        wv = w_v[sl]
        rowb = lax.shift_right_logical(dv, 4)
        lane = lax.bitwise_and(dv, 15)
        for bb in range(B_HALF):
            bvec = jnp.full((16,), bb, jnp.int32)
            g = plsc.load_gather(x_v, [bvec, sv])
            plsc.addupdate_scatter(acc_v, [rowb + (bb * ROWS_PER_B), lane],
                                   g * wv)
        return carry

    lax.fori_loop(0, VECS, step, 0)

    plsc.subcore_barrier()

    @pl.when(s == 0)
    def _seed():
        pltpu.sync_copy(acc_v, shared_acc)

    plsc.subcore_barrier()

    @pl.when(s != 0)
    def _accum():
        pltpu.sync_copy(acc_v, shared_acc.at[rowidx_v], add=True)

    plsc.subcore_barrier()

    n0 = s * N_PER_TILE
    pltpu.sync_copy(bias_hbm.at[pl.ds(n0, N_PER_TILE)], bias_v)
    for bb in range(B_HALF):
        r0 = bb * ROWS_PER_B + s * FROWS
        pltpu.sync_copy(shared_acc.at[pl.ds(r0, FROWS)], fbuf_v)
        for j in range(FROWS):
            v = fbuf_v[j] + bias_v[pl.ds(j * 16, 16)]
            obuf_v[bb, pl.ds(j * 16, 16)] = jnp.maximum(v, 0.0)
    pltpu.sync_copy(obuf_v,
                    out_hbm.at[pl.ds(c * B_HALF, B_HALF), pl.ds(n0, N_PER_TILE)])


def kernel(inputs, src, dst, adj_vals, w, b):
    xpad = jnp.zeros((B, N_PAD), jnp.float32).at[:, :N_NODES].set(inputs)
    weff = adj_vals * w
    bpad = jnp.zeros((N_PAD,), jnp.float32).at[:N_NODES].set(b)
    rowidx = lax.iota(jnp.int32, ROWS)
    zeros = jnp.zeros((ROWS, 16), jnp.float32)

    fn = pl.kernel(
        _gnn_body,
        mesh=plsc.VectorSubcoreMesh(core_axis_name="c", subcore_axis_name="s"),
        out_type=jax.ShapeDtypeStruct((B, N_PAD), jnp.float32),
        scratch_types=[
            pltpu.VMEM((B_HALF, N_PAD), jnp.float32),
            pltpu.VMEM((ROWS, 16), jnp.float32),
            pltpu.VMEM((E_PER_TILE,), jnp.int32),
            pltpu.VMEM((E_PER_TILE,), jnp.int32),
            pltpu.VMEM((E_PER_TILE,), jnp.float32),
            pltpu.VMEM((ROWS,), jnp.int32),
            pltpu.VMEM((FROWS, 16), jnp.float32),
            pltpu.VMEM((B_HALF, N_PER_TILE), jnp.float32),
            pltpu.VMEM((N_PER_TILE,), jnp.float32),
            pltpu.VMEM_SHARED((ROWS, 16), jnp.float32),
        ],
    )
    out = fn(xpad, src, dst, weff, bpad, rowidx, zeros)
    return out[:, :N_NODES]


# speedup vs baseline: 14.0599x; 14.0599x over previous
import jax
import jax.numpy as jnp
from jax import lax
from jax.experimental import pallas as pl
from jax.experimental.pallas import tpu as pltpu
from jax.experimental.pallas import tpu_sc as plsc

N_NODES = 10000
N_PAD = 10240
B = 8
B_HALF = 4
E = 160000
TILES = 16
E_PER_TILE = E // TILES
E_CHUNK = 2000
E_ROUNDS = E_PER_TILE // E_CHUNK
CVECS = E_CHUNK // 16
ACC_WORDS = B_HALF * N_PAD
N_PER_TILE = N_PAD // TILES
FVECS = N_PER_TILE // 16


def _gnn_body(x_hbm, src_hbm, dst_hbm, w_hbm, bias_hbm,
              out_hbm,
              x_v, acc_v, src_v, dst_v, w_v, fbuf_v, obuf_v, bias_v,
              shared_pl):
    c = lax.axis_index("c")
    s = lax.axis_index("s")

    pltpu.sync_copy(x_hbm.at[c], x_v)

    zv = jnp.zeros((16,), jnp.float32)

    def zstep(j, carry):
        base = j * 128
        for u in range(8):
            acc_v[pl.ds(base + u * 16, 16)] = zv
        return carry

    lax.fori_loop(0, ACC_WORDS // 128, zstep, 0)

    def round_(r, carry):
        e0 = s * E_PER_TILE + r * E_CHUNK
        pltpu.sync_copy(src_hbm.at[pl.ds(e0, E_CHUNK)], src_v)
        pltpu.sync_copy(dst_hbm.at[pl.ds(e0, E_CHUNK)], dst_v)
        pltpu.sync_copy(w_hbm.at[pl.ds(e0, E_CHUNK)], w_v)

        def step(i, carry2):
            sl = pl.ds(i * 16, 16)
            sv = src_v[sl]
            dv = dst_v[sl]
            wv = w_v[sl]
            for bb in range(B_HALF):
                g = plsc.load_gather(x_v, [sv + bb * N_PAD])
                plsc.addupdate_scatter(acc_v, [dv + bb * N_PAD], g * wv)
            return carry2

        lax.fori_loop(0, CVECS, step, 0)
        return carry

    lax.fori_loop(0, E_ROUNDS, round_, 0)

    n0 = s * N_PER_TILE
    pltpu.sync_copy(bias_hbm.at[pl.ds(n0, N_PER_TILE)], bias_v)
    for bb in range(B_HALF):
        plsc.subcore_barrier()
        pltpu.sync_copy(acc_v.at[pl.ds(bb * N_PAD, N_PAD)], shared_pl.at[s])
        plsc.subcore_barrier()
        pltpu.sync_copy(shared_pl.at[:, pl.ds(n0, N_PER_TILE)], fbuf_v)

        def fstep(j, carry, bb=bb):
            sl = pl.ds(j * 16, 16)
            v = fbuf_v[0, sl]
            for t in range(1, TILES):
                v = v + fbuf_v[t, sl]
            obuf_v[bb, sl] = jnp.maximum(v + bias_v[sl], 0.0)
            return carry

        lax.fori_loop(0, FVECS, fstep, 0)

    pltpu.sync_copy(obuf_v,
                    out_hbm.at[pl.ds(c * B_HALF, B_HALF), pl.ds(n0, N_PER_TILE)])


def kernel(inputs, src, dst, adj_vals, w, b):
    xpad = jnp.zeros((B, N_PAD), jnp.float32).at[:, :N_NODES].set(inputs)
    xflat = xpad.reshape(2, B_HALF * N_PAD)
    weff = adj_vals * w
    bpad = jnp.zeros((N_PAD,), jnp.float32).at[:N_NODES].set(b)

    fn = pl.kernel(
        _gnn_body,
        mesh=plsc.VectorSubcoreMesh(core_axis_name="c", subcore_axis_name="s"),
        out_type=jax.ShapeDtypeStruct((B, N_PAD), jnp.float32),
        compiler_params=pltpu.CompilerParams(needs_layout_passes=False),
        scratch_types=[
            pltpu.VMEM((B_HALF * N_PAD,), jnp.float32),
            pltpu.VMEM((ACC_WORDS,), jnp.float32),
            pltpu.VMEM((E_CHUNK,), jnp.int32),
            pltpu.VMEM((E_CHUNK,), jnp.int32),
            pltpu.VMEM((E_CHUNK,), jnp.float32),
            pltpu.VMEM((TILES, N_PER_TILE), jnp.float32),
            pltpu.VMEM((B_HALF, N_PER_TILE), jnp.float32),
            pltpu.VMEM((N_PER_TILE,), jnp.float32),
            pltpu.VMEM_SHARED((TILES, N_PAD), jnp.float32),
        ],
    )
    out = fn(xflat, src, dst, weff, bpad)
    return out[:, :N_NODES]


# speedup vs baseline: 17.8975x; 1.2729x over previous
import jax
import jax.numpy as jnp
from jax import lax
from jax.experimental import pallas as pl
from jax.experimental.pallas import tpu as pltpu
from jax.experimental.pallas import tpu_sc as plsc

N_NODES = 10000
N_PAD = 10240
B = 8
B_HALF = 4
E = 160000
TILES = 16
E_PER_TILE = E // TILES
VECS = E_PER_TILE // 16
UNROLL = 5
ACC_ROWS = B_HALF * N_PAD // 16
ROWS_PER_B = N_PAD // 16
N_PER_TILE = N_PAD // TILES
FVECS = N_PER_TILE // 16
N_LAST = N_NODES - 15 * N_PER_TILE


def _gnn_body(x_hbm, src_hbm, dst_hbm, w_hbm, bias_hbm, rowidx_hbm,
              out_hbm,
              x_v, acc_v, src_v, dst_v, w_v, rowidx_v, fbuf_v, obuf_v, bias_v,
              shared_acc, sem):
    c = lax.axis_index("c")
    s = lax.axis_index("s")

    e0 = s * E_PER_TILE
    h_x = pltpu.async_copy(x_hbm.at[c], x_v, sem)
    h_s = pltpu.async_copy(src_hbm.at[pl.ds(e0, E_PER_TILE)], src_v, sem)
    h_d = pltpu.async_copy(dst_hbm.at[pl.ds(e0, E_PER_TILE)], dst_v, sem)
    h_w = pltpu.async_copy(w_hbm.at[pl.ds(e0, E_PER_TILE)], w_v, sem)
    h_r = pltpu.async_copy(rowidx_hbm, rowidx_v, sem)

    zv = jnp.zeros((16,), jnp.float32)

    def zstep(j, carry):
        base = j * 8
        for u in range(8):
            acc_v[base + u] = zv
        return carry

    lax.fori_loop(0, ACC_ROWS // 8, zstep, 0)

    stripe = ACC_ROWS // TILES
    pltpu.sync_copy(acc_v.at[pl.ds(s * stripe, stripe)],
                    shared_acc.at[pl.ds(s * stripe, stripe)])
    plsc.subcore_barrier()

    h_x.wait()
    h_s.wait()
    h_d.wait()
    h_w.wait()
    h_r.wait()

    def step(i, carry):
        for u in range(UNROLL):
            sl = pl.ds((i * UNROLL + u) * 16, 16)
            sv = src_v[sl]
            dv = dst_v[sl]
            wv = w_v[sl]
            row = lax.shift_right_logical(dv, 4)
            lane = lax.bitwise_and(dv, 15)
            for bb in range(B_HALF):
                g = plsc.load_gather(x_v, [sv + bb * N_NODES])
                plsc.addupdate_scatter(acc_v, [row + bb * ROWS_PER_B, lane],
                                       g * wv)
        return carry

    lax.fori_loop(0, VECS // UNROLL, step, 0)

    pltpu.sync_copy(acc_v, shared_acc.at[rowidx_v], add=True)
    plsc.subcore_barrier()

    n0 = s * N_PER_TILE

    pltpu.sync_copy(bias_hbm.at[pl.ds(n0, N_PER_TILE)], bias_v)

    for bb in range(B_HALF):
        pltpu.sync_copy(
            shared_acc.at[pl.ds(bb * ROWS_PER_B + s * FVECS, FVECS)],
            fbuf_v.at[pl.ds(bb * FVECS, FVECS)])

    def fstep(j, carry):
        sl = pl.ds(j * 16, 16)
        for bb in range(B_HALF):
            v = fbuf_v[bb * FVECS + j]
            obuf_v[bb, sl] = jnp.maximum(v + bias_v[sl], 0.0)
        return carry

    lax.fori_loop(0, FVECS, fstep, 0)

    pltpu.sync_copy(obuf_v,
                    out_hbm.at[pl.ds(c * B_HALF, B_HALF),
                               pl.ds(n0, N_PER_TILE)])


def kernel(inputs, src, dst, adj_vals, w, b):
    xflat = inputs.reshape(2, B_HALF * N_NODES)
    weff = adj_vals * w
    bpad = jnp.zeros((N_PAD,), jnp.float32).at[:N_NODES].set(b)
    rowidx = lax.iota(jnp.int32, ACC_ROWS)

    fn = pl.kernel(
        _gnn_body,
        mesh=plsc.VectorSubcoreMesh(core_axis_name="c", subcore_axis_name="s"),
        out_type=jax.ShapeDtypeStruct((B, N_PAD), jnp.float32),
        compiler_params=pltpu.CompilerParams(needs_layout_passes=False,
                                             use_tc_tiling_on_sc=False),
        scratch_types=[
            pltpu.VMEM((B_HALF * N_NODES,), jnp.float32),
            pltpu.VMEM((ACC_ROWS, 16), jnp.float32),
            pltpu.VMEM((E_PER_TILE,), jnp.int32),
            pltpu.VMEM((E_PER_TILE,), jnp.int32),
            pltpu.VMEM((E_PER_TILE,), jnp.float32),
            pltpu.VMEM((ACC_ROWS,), jnp.int32),
            pltpu.VMEM((B_HALF * FVECS, 16), jnp.float32),
            pltpu.VMEM((B_HALF, N_PER_TILE), jnp.float32),
            pltpu.VMEM((N_PER_TILE,), jnp.float32),
            pltpu.VMEM_SHARED((ACC_ROWS, 16), jnp.float32),
            pltpu.SemaphoreType.DMA,
        ],
    )
    out = fn(xflat, src, dst, weff, bpad, rowidx)
    return out[:, :N_NODES]


# speedup vs baseline: 24.6982x; 1.3800x over previous
import jax
import jax.numpy as jnp
from jax import lax
from jax.experimental import pallas as pl
from jax.experimental.pallas import tpu as pltpu
from jax.experimental.pallas import tpu_sc as plsc

N_NODES = 10000
N_PAD = 10240
B = 8
B_HALF = 4
E = 160000
TILES = 16
E_PER_TILE = E // TILES
VECS = E_PER_TILE // 16
UNROLL = 5
ACC_ROWS = B_HALF * N_PAD // 16
ROWS_PER_B = N_PAD // 16
N_PER_TILE = N_PAD // TILES
FVECS = N_PER_TILE // 16
N_LAST = N_NODES - 15 * N_PER_TILE


def _gnn_body(x_hbm, src_hbm, dst_hbm, w_hbm, bias_hbm, rowidx_hbm,
              out_hbm,
              x_v, acc_v, src_v, dst_v, w_v, rowidx_v, fbuf_v, obuf_v, bias_v,
              shared_acc, shared_x, sem):
    c = lax.axis_index("c")
    s = lax.axis_index("s")

    e0 = s * E_PER_TILE
    h_s = pltpu.async_copy(src_hbm.at[pl.ds(e0, E_PER_TILE)], src_v, sem)
    h_d = pltpu.async_copy(dst_hbm.at[pl.ds(e0, E_PER_TILE)], dst_v, sem)
    h_w = pltpu.async_copy(w_hbm.at[pl.ds(e0, E_PER_TILE)], w_v, sem)
    h_r = pltpu.async_copy(rowidx_hbm, rowidx_v, sem)

    @pl.when(s == 0)
    def _stage_x():
        pltpu.sync_copy(x_hbm.at[c], shared_x)

    zv = jnp.zeros((16,), jnp.float32)

    @plsc.parallel_loop(0, ACC_ROWS // 8, unroll=4)
    def zstep(j):
        base = j * 8
        for u in range(8):
            acc_v[base + u] = zv

    stripe = ACC_ROWS // TILES
    pltpu.sync_copy(acc_v.at[pl.ds(s * stripe, stripe)],
                    shared_acc.at[pl.ds(s * stripe, stripe)])
    plsc.subcore_barrier()

    pltpu.sync_copy(shared_x, x_v)

    h_s.wait()
    h_d.wait()
    h_w.wait()
    h_r.wait()

    @plsc.parallel_loop(0, VECS, unroll=UNROLL)
    def step(i):
        sl = pl.ds(i * 16, 16)
        sv = src_v[sl]
        dv = dst_v[sl]
        wv = w_v[sl]
        row = lax.shift_right_logical(dv, 4)
        lane = lax.bitwise_and(dv, 15)
        for bb in range(B_HALF):
            g = plsc.load_gather(x_v, [sv + bb * N_NODES])
            plsc.addupdate_scatter(acc_v, [row + bb * ROWS_PER_B, lane],
                                   g * wv)

    pltpu.sync_copy(acc_v, shared_acc.at[rowidx_v], add=True)
    plsc.subcore_barrier()

    n0 = s * N_PER_TILE

    pltpu.sync_copy(bias_hbm.at[pl.ds(n0, N_PER_TILE)], bias_v)

    for bb in range(B_HALF):
        pltpu.sync_copy(
            shared_acc.at[pl.ds(bb * ROWS_PER_B + s * FVECS, FVECS)],
            fbuf_v.at[pl.ds(bb * FVECS, FVECS)])

    @plsc.parallel_loop(0, FVECS, unroll=4)
    def fstep(j):
        sl = pl.ds(j * 16, 16)
        for bb in range(B_HALF):
            v = fbuf_v[bb * FVECS + j]
            obuf_v[bb, sl] = jnp.maximum(v + bias_v[sl], 0.0)

    pltpu.sync_copy(obuf_v,
                    out_hbm.at[pl.ds(c * B_HALF, B_HALF),
                               pl.ds(n0, N_PER_TILE)])


def kernel(inputs, src, dst, adj_vals, w, b):
    xflat = inputs.reshape(2, B_HALF * N_NODES)
    weff = adj_vals * w
    bpad = jnp.zeros((N_PAD,), jnp.float32).at[:N_NODES].set(b)
    rowidx = lax.iota(jnp.int32, ACC_ROWS)

    fn = pl.kernel(
        _gnn_body,
        mesh=plsc.VectorSubcoreMesh(core_axis_name="c", subcore_axis_name="s"),
        out_type=jax.ShapeDtypeStruct((B, N_PAD), jnp.float32),
        compiler_params=pltpu.CompilerParams(needs_layout_passes=False,
                                             use_tc_tiling_on_sc=False),
        scratch_types=[
            pltpu.VMEM((B_HALF * N_NODES,), jnp.float32),
            pltpu.VMEM((ACC_ROWS, 16), jnp.float32),
            pltpu.VMEM((E_PER_TILE,), jnp.int32),
            pltpu.VMEM((E_PER_TILE,), jnp.int32),
            pltpu.VMEM((E_PER_TILE,), jnp.float32),
            pltpu.VMEM((ACC_ROWS,), jnp.int32),
            pltpu.VMEM((B_HALF * FVECS, 16), jnp.float32),
            pltpu.VMEM((B_HALF, N_PER_TILE), jnp.float32),
            pltpu.VMEM((N_PER_TILE,), jnp.float32),
            pltpu.VMEM_SHARED((ACC_ROWS, 16), jnp.float32),
            pltpu.VMEM_SHARED((B_HALF * N_NODES,), jnp.float32),
            pltpu.SemaphoreType.DMA,
        ],
    )
    out = fn(xflat, src, dst, weff, bpad, rowidx)
    return out[:, :N_NODES]
